# baseline (speedup 1.0000x reference)
import jax
import jax.numpy as jnp
from jax.experimental import pallas as pl
from jax.experimental.pallas import tpu as pltpu

CHUNK = 2048
SLAB = 128


def _rpq_kernel(x_ref, p_ref, ct_ref, out_ref):
    G, Lb, D = x_ref.shape
    x = x_ref[...].reshape(G * Lb, D)
    t = jnp.dot(x, p_ref[...], preferred_element_type=jnp.float32)
    nrm = jnp.sqrt(jnp.sum(t * t, axis=-1, keepdims=True))
    tn = t / jnp.maximum(nrm, 1e-12)
    L = x.shape[0]
    K = ct_ref.shape[1]

    lane_f = jax.lax.broadcasted_iota(
        jnp.int32, (L, SLAB), 1).astype(jnp.float32)
    bv = jnp.full((L, SLAB), -jnp.inf, jnp.float32)
    bg = jnp.zeros((L, SLAB), jnp.float32)
    for c in range(K // CHUNK):
        s = jnp.dot(tn, ct_ref[:, c * CHUNK:(c + 1) * CHUNK],
                    preferred_element_type=jnp.float32)
        for g in range(CHUNK // SLAB):
            gg = c * (CHUNK // SLAB) + g
            sg = s[:, g * SLAB:(g + 1) * SLAB]
            if gg == 0:
                bv = sg
            else:
                m = sg > bv
                bv = jnp.maximum(bv, sg)
                bg = jnp.where(m, jnp.float32(gg), bg)
    cmax = jnp.max(bv, axis=1, keepdims=True)
    gidx = bg * SLAB + lane_f
    cand = jnp.where(bv == cmax, gidx, jnp.float32(K))
    best_idx = jnp.min(cand, axis=1, keepdims=True)
    out_ref[...] = best_idx.astype(jnp.int32).reshape(G, Lb, 1)


@jax.jit
def kernel(masked_target_values, project_mat, codebook_norm):
    B, L, D = masked_target_values.shape
    K, C = codebook_norm.shape
    ct = codebook_norm.T

    GRP = 2
    out = pl.pallas_call(
        _rpq_kernel,
        grid=(B // GRP,),
        in_specs=[
            pl.BlockSpec((GRP, L, D), lambda b: (b, 0, 0)),
            pl.BlockSpec((D, C), lambda b: (0, 0)),
            pl.BlockSpec((C, K), lambda b: (0, 0)),
        ],
        out_specs=pl.BlockSpec((GRP, L, 1), lambda b: (b, 0, 0)),
        out_shape=jax.ShapeDtypeStruct((B, L, 1), jnp.int32),
    )(masked_target_values, project_mat, ct)
    return out[:, :, 0]


# speedup vs baseline: 1.0082x; 1.0082x over previous
import jax
import jax.numpy as jnp
from jax.experimental import pallas as pl

CHUNK = 2048
SLAB = 128


def _rpq_kernel(x_ref, p_ref, ct_ref, out_ref):
    G, Lb, D = x_ref.shape
    x = x_ref[...].reshape(G * Lb, D)
    t = jnp.dot(x, p_ref[...], preferred_element_type=jnp.float32)
    nrm = jnp.sqrt(jnp.sum(t * t, axis=-1, keepdims=True))
    tn = t / jnp.maximum(nrm, 1e-12)
    L = x.shape[0]
    K = ct_ref.shape[1]

    lane_f = jax.lax.broadcasted_iota(
        jnp.int32, (L, SLAB), 1).astype(jnp.float32)
    bv = jnp.full((L, SLAB), -jnp.inf, jnp.float32)
    bg = jnp.zeros((L, SLAB), jnp.float32)
    for c in range(K // CHUNK):
        s = jnp.dot(tn, ct_ref[:, c * CHUNK:(c + 1) * CHUNK],
                    preferred_element_type=jnp.float32)
        for g in range(CHUNK // SLAB):
            gg = c * (CHUNK // SLAB) + g
            sg = s[:, g * SLAB:(g + 1) * SLAB]
            if gg == 0:
                bv = sg
            else:
                m = sg > bv
                bv = jnp.maximum(bv, sg)
                bg = jnp.where(m, jnp.float32(gg), bg)
    cmax = jnp.max(bv, axis=1, keepdims=True)
    gidx = bg * SLAB + lane_f
    cand = jnp.where(bv == cmax, gidx, jnp.float32(K))
    best_idx = jnp.min(cand, axis=1, keepdims=True)
    out_ref[...] = best_idx.astype(jnp.int32).reshape(G, Lb, 1)


@jax.jit
def kernel(masked_target_values, project_mat, codebook_norm):
    B, L, D = masked_target_values.shape
    K, C = codebook_norm.shape
    ct = codebook_norm.T

    GRP = 1
    out = pl.pallas_call(
        _rpq_kernel,
        grid=(B // GRP,),
        in_specs=[
            pl.BlockSpec((GRP, L, D), lambda b: (b, 0, 0)),
            pl.BlockSpec((D, C), lambda b: (0, 0)),
            pl.BlockSpec((C, K), lambda b: (0, 0)),
        ],
        out_specs=pl.BlockSpec((GRP, L, 1), lambda b: (b, 0, 0)),
        out_shape=jax.ShapeDtypeStruct((B, L, 1), jnp.int32),
    )(masked_target_values, project_mat, ct)
    return out[:, :, 0]
